# Initial kernel scaffold; baseline (speedup 1.0000x reference)
import jax
import jax.numpy as jnp
from jax.experimental import pallas as pl


def kernel(x, edges, A_init, W, b, temperature):
    raise NotImplementedError("write your pallas kernel here")


# speedup vs baseline: 1.7603x; 1.7603x over previous
import functools

import jax
import jax.numpy as jnp
from jax import lax
from jax.experimental import pallas as pl
from jax.experimental.pallas import tpu as pltpu

_N = 10000
_E = 320000
_DF = 128
_NA = 500
_NAP = 512
_NCHUNK = 4
_K = 16
_RB = 1000


def _nt(a, b):
    return lax.dot_general(a, b, (((1,), (1,)), ((), ())),
                           precision=lax.Precision.HIGHEST,
                           preferred_element_type=jnp.float32)


def _bf(a):
    return a.astype(jnp.bfloat16)


def _bfdot(a, b):
    return jnp.dot(_bf(a), _bf(b), preferred_element_type=jnp.float32)


def _bfnt(a, b):
    return lax.dot_general(_bf(a), _bf(b), (((1,), (1,)), ((), ())),
                           preferred_element_type=jnp.float32)


def _emb_prob_body(x_ref, ax_ref, w_ref, b_ref, t_ref,
                   xemb_ref, p0_ref, p1_ref, p2_ref, p3_ref,
                   aemb_scr, bb_scr):
    i = pl.program_id(0)
    w = w_ref[...]
    b = b_ref[...]

    @pl.when(i == 0)
    def _():
        ae = _bfdot(ax_ref[...], w) + b
        aemb_scr[...] = ae
        bb_scr[...] = _nt(jnp.ones((1, _DF), jnp.float32), ae * ae)

    xe = _bfdot(x_ref[...], w) + b
    xemb_ref[...] = xe
    aa = jnp.sum(xe * xe, axis=1, keepdims=True)
    ab = _bfnt(xe, aemb_scr[...])
    d2 = aa + bb_scr[...] - 2.0 * ab
    dist = jnp.sqrt(jnp.clip(d2, 0.0, None) + 1e-12)
    t = jnp.exp(jnp.clip(t_ref[0], -5.0, 5.0))
    prob = jnp.exp(-dist * t)
    p0_ref[...] = prob[:, 0:128]
    p1_ref[...] = prob[:, 128:256]
    p2_ref[...] = prob[:, 256:384]
    p3_ref[...] = prob[:, 384:512]


def _emb_prob(x, ax, w, b, t):
    grid = _N // _RB
    out_shapes = (
        jax.ShapeDtypeStruct((_N, _DF), jnp.float32),
        *(jax.ShapeDtypeStruct((_N, 128), jnp.float32) for _ in range(_NCHUNK)),
    )
    chunk_spec = pl.BlockSpec((_RB, 128), lambda i: (i, 0))
    return pl.pallas_call(
        _emb_prob_body,
        grid=(grid,),
        in_specs=[
            pl.BlockSpec((_RB, _DF), lambda i: (i, 0)),
            pl.BlockSpec((_NAP, _DF), lambda i: (0, 0)),
            pl.BlockSpec((_DF, _DF), lambda i: (0, 0)),
            pl.BlockSpec((1, _DF), lambda i: (0, 0)),
            pl.BlockSpec(memory_space=pltpu.SMEM),
        ],
        out_specs=(
            pl.BlockSpec((_RB, _DF), lambda i: (i, 0)),
            chunk_spec, chunk_spec, chunk_spec, chunk_spec,
        ),
        out_shape=out_shapes,
        scratch_shapes=[
            pltpu.VMEM((_NAP, _DF), jnp.float32),
            pltpu.VMEM((1, _NAP), jnp.float32),
        ],
    )(x, ax, w, b, t)


def _topk_body(p0_ref, p1_ref, p2_ref, p3_ref,
               a0_ref, a1_ref, a2_ref, a3_ref,
               deg_ref, q_ref,
               ehat_ref, lp_ref):
    deg = jnp.clip(deg_ref[...], 1.0, None)
    chunks = []
    for pr, ar in ((p0_ref, a0_ref), (p1_ref, a1_ref),
                   (p2_ref, a2_ref), (p3_ref, a3_ref)):
        agg = pr[...] + ar[...]
        chunks.append(agg)
    merge = jnp.concatenate(chunks, axis=1) / deg
    logprobs = -jnp.log(jnp.clip(merge, 1e-12, None))
    gl = jnp.log(-jnp.log(q_ref[...]))
    cols = lax.broadcasted_iota(jnp.int32, (_RB, _NAP), 1)
    score = jnp.where(cols < _NA, gl - logprobs, -3e38)

    eh = jnp.zeros((_RB, _NAP), jnp.float32)
    lps = []
    for _ in range(_K):
        m = jnp.max(score, axis=1, keepdims=True)
        ismax = score >= m
        idx = jnp.min(jnp.where(ismax, cols, _NAP), axis=1, keepdims=True)
        sel = cols == idx
        eh = jnp.where(sel, 1.0, eh)
        lps.append(m)
        score = jnp.where(sel, -3e38, score)
    ehat_ref[...] = eh[:, :_NA]
    lp_ref[...] = jnp.concatenate(lps, axis=1)


def _topk(pchunks, achunks, deg, q):
    grid = _N // _RB
    chunk_spec = pl.BlockSpec((_RB, 128), lambda i: (i, 0))
    return pl.pallas_call(
        _topk_body,
        grid=(grid,),
        in_specs=[chunk_spec] * 8 + [
            pl.BlockSpec((_RB, 1), lambda i: (i, 0)),
            pl.BlockSpec((_RB, _NAP), lambda i: (i, 0)),
        ],
        out_specs=(
            pl.BlockSpec((_RB, _NA), lambda i: (i, 0)),
            pl.BlockSpec((_RB, _K), lambda i: (i, 0)),
        ),
        out_shape=(
            jax.ShapeDtypeStruct((_N, _NA), jnp.float32),
            jax.ShapeDtypeStruct((_N, _K), jnp.float32),
        ),
    )(*pchunks, *achunks, deg, q)


def kernel(x, edges, A_init, W, b, temperature):
    del edges
    anchor_idx = jax.random.choice(jax.random.key(1), _N, shape=(_NA,),
                                   replace=False)
    q = jax.random.uniform(jax.random.key(2), (_N, _NA), dtype=jnp.float32)
    q = q + 1e-12
    qp = jnp.pad(q, ((0, 0), (0, _NAP - _NA)), constant_values=0.5)

    ax = jnp.zeros((_NAP, _DF), jnp.float32).at[:_NA].set(x[anchor_idx])
    t = jnp.reshape(temperature, (1,)).astype(jnp.float32)
    b2 = jnp.reshape(b, (1, _DF))

    xemb, p0, p1, p2, p3 = _emb_prob(x, ax, W, b2, t)

    prob = jnp.concatenate([p0, p1, p2, p3], axis=1)
    src = A_init[0]
    dst = A_init[1]
    agg = jax.ops.segment_sum(prob[dst], src, num_segments=_N)
    achunks = tuple(agg[:, c * 128:(c + 1) * 128] for c in range(_NCHUNK))
    deg = 1.0 + jax.ops.segment_sum(jnp.ones((_E,), jnp.float32), src,
                                    num_segments=_N)
    deg = jnp.reshape(deg, (_N, 1))

    ehat, lp = _topk((p0, p1, p2, p3), achunks, deg, qp)
    return (xemb, ehat, lp)


# speedup vs baseline: 3.7984x; 2.1579x over previous
import functools

import jax
import jax.numpy as jnp
from jax import lax
from jax.experimental import pallas as pl
from jax.experimental.pallas import tpu as pltpu
from jax.experimental.pallas import tpu_sc as plsc

_N = 10000
_E = 320000
_DF = 128
_NA = 500
_NAP = 512
_NCHUNK = 4
_CW = 128
_K = 16
_RB = 1000


def _nt(a, b):
    return lax.dot_general(a, b, (((1,), (1,)), ((), ())),
                           precision=lax.Precision.HIGHEST,
                           preferred_element_type=jnp.float32)


def _bf(a):
    return a.astype(jnp.bfloat16)


def _bfdot(a, b):
    return jnp.dot(_bf(a), _bf(b), preferred_element_type=jnp.float32)


def _bfnt(a, b):
    return lax.dot_general(_bf(a), _bf(b), (((1,), (1,)), ((), ())),
                           preferred_element_type=jnp.float32)


def _emb_prob_body(x_ref, ax_ref, w_ref, b_ref, t_ref,
                   xemb_ref, p0_ref, p1_ref, p2_ref, p3_ref,
                   aemb_scr, bb_scr):
    i = pl.program_id(0)
    w = w_ref[...]
    b = b_ref[...]

    @pl.when(i == 0)
    def _():
        ae = _bfdot(ax_ref[...], w) + b
        aemb_scr[...] = ae
        bb_scr[...] = _nt(jnp.ones((1, _DF), jnp.float32), ae * ae)

    xe = _bfdot(x_ref[...], w) + b
    xemb_ref[...] = xe
    aa = jnp.sum(xe * xe, axis=1, keepdims=True)
    ab = _bfnt(xe, aemb_scr[...])
    d2 = aa + bb_scr[...] - 2.0 * ab
    dist = jnp.sqrt(jnp.clip(d2, 0.0, None) + 1e-12)
    t = jnp.exp(jnp.clip(t_ref[0], -5.0, 5.0))
    prob = jnp.exp(-dist * t)
    for ci, pr in enumerate((p0_ref, p1_ref, p2_ref, p3_ref)):
        pr[...] = prob[:, ci * 128:(ci + 1) * 128]


def _emb_prob(x, ax, w, b, t):
    grid = _N // _RB
    out_shapes = (
        jax.ShapeDtypeStruct((_N, _DF), jnp.float32),
        *(jax.ShapeDtypeStruct((_N, _CW), jnp.float32) for _ in range(_NCHUNK)),
    )
    chunk_spec = pl.BlockSpec((_RB, _CW), lambda i: (i, 0))
    return pl.pallas_call(
        _emb_prob_body,
        grid=(grid,),
        in_specs=[
            pl.BlockSpec((_RB, _DF), lambda i: (i, 0)),
            pl.BlockSpec((_NAP, _DF), lambda i: (0, 0)),
            pl.BlockSpec((_DF, _DF), lambda i: (0, 0)),
            pl.BlockSpec((1, _DF), lambda i: (0, 0)),
            pl.BlockSpec(memory_space=pltpu.SMEM),
        ],
        out_specs=(
            pl.BlockSpec((_RB, _DF), lambda i: (i, 0)),
            chunk_spec, chunk_spec, chunk_spec, chunk_spec,
        ),
        out_shape=out_shapes,
        scratch_shapes=[
            pltpu.VMEM((_NAP, _DF), jnp.float32),
            pltpu.VMEM((1, _NAP), jnp.float32),
        ],
    )(x, ax, w, b, t)


_NW = 32
_BE = 128
_NBATCH = -(-_E // (_NW * _BE))
_EPW = _BE * _NBATCH
_EPAD = _NW * _EPW
_NPAD = 10240
_STRIPE = _NPAD // 16


def _sc_scatter(pchunks, srcp, dstp, zrows, ones128):
    mesh = plsc.VectorSubcoreMesh(core_axis_name="c", subcore_axis_name="s")

    @functools.partial(
        pl.kernel,
        out_type=tuple(jax.ShapeDtypeStruct((_NPAD, _CW), jnp.float32)
                       for _ in range(2 * (_NCHUNK + 1))),
        mesh=mesh,
        scratch_types=[
            pltpu.VMEM((_BE,), jnp.int32),
            pltpu.VMEM((_BE,), jnp.int32),
            pltpu.VMEM((_BE, _CW), jnp.float32),
            pltpu.VMEM((_BE, _CW), jnp.float32),
            pltpu.VMEM_SHARED((_NPAD, _CW), jnp.float32),
            pltpu.SemaphoreType.DMA,
        ],
    )
    def k(p0h, p1h, p2h, p3h, srch, dsth, zrh, onesh,
          a00, a01, a02, a03, ad0, a10, a11, a12, a13, ad1,
          idxd, idxs, rows, ones_v, agg_s, sem):
        aggp = (a00, a01, a02, a03, ad0, a10, a11, a12, a13, ad1)
        nout = _NCHUNK + 1
        c = lax.axis_index("c")
        s = lax.axis_index("s")
        wid = s * 2 + c
        ebase = wid * _EPW
        rslice = pl.ds(s * _STRIPE, _STRIPE)
        pltpu.sync_copy(onesh, ones_v)
        for ci in range(_NCHUNK + 1):
            ph = (p0h, p1h, p2h, p3h)[ci] if ci < _NCHUNK else None
            pltpu.sync_copy(zrh.at[rslice], agg_s.at[rslice])
            plsc.subcore_barrier()

            def body(i, carry):
                off = ebase + i * _BE
                pltpu.sync_copy(srch.at[pl.ds(off, _BE)], idxs)
                if ph is not None:
                    pltpu.sync_copy(dsth.at[pl.ds(off, _BE)], idxd)
                    pltpu.async_copy(ph.at[idxd], rows, sem).wait()
                    pltpu.sync_copy(rows, agg_s.at[idxs], add=True)
                else:
                    pltpu.sync_copy(ones_v, agg_s.at[idxs], add=True)
                return carry

            lax.fori_loop(0, _NBATCH, body, 0)
            plsc.subcore_barrier()
            for cc in range(2):
                @pl.when(c == cc)
                def _(ci=ci, cc=cc):
                    pltpu.sync_copy(agg_s.at[rslice],
                                    aggp[cc * nout + ci].at[rslice])

    return k(*pchunks, srcp, dstp, zrows, ones128)


def _topk_body(p0_ref, p1_ref, p2_ref, p3_ref,
               a00_ref, a01_ref, a02_ref, a03_ref, ad0_ref,
               a10_ref, a11_ref, a12_ref, a13_ref, ad1_ref,
               q_ref,
               ehat_ref, lp_ref):
    deg = 1.0 + ad0_ref[:, 0:1] + ad1_ref[:, 0:1]
    a0 = (a00_ref, a01_ref, a02_ref, a03_ref)
    a1 = (a10_ref, a11_ref, a12_ref, a13_ref)
    chunks = []
    for ci, pr in enumerate((p0_ref, p1_ref, p2_ref, p3_ref)):
        agg = pr[...] + a0[ci][...] + a1[ci][...]
        chunks.append(agg)
    merge = jnp.concatenate(chunks, axis=1) / deg
    logprobs = -jnp.log(jnp.clip(merge, 1e-12, None))
    gl = jnp.log(-jnp.log(q_ref[...]))
    cols = lax.broadcasted_iota(jnp.int32, (_RB, _NAP), 1)
    score = jnp.where(cols < _NA, gl - logprobs, -3e38)

    eh = jnp.zeros((_RB, _NAP), jnp.float32)
    lps = []
    for _ in range(_K):
        m = jnp.max(score, axis=1, keepdims=True)
        ismax = score >= m
        idx = jnp.min(jnp.where(ismax, cols, _NAP), axis=1, keepdims=True)
        sel = cols == idx
        eh = jnp.where(sel, 1.0, eh)
        lps.append(m)
        score = jnp.where(sel, -3e38, score)
    ehat_ref[...] = eh[:, :_NA]
    lp_ref[...] = jnp.concatenate(lps, axis=1)


def _topk(pchunks, aggp, q):
    grid = _N // _RB
    chunk_spec = pl.BlockSpec((_RB, _CW), lambda i: (i, 0))
    return pl.pallas_call(
        _topk_body,
        grid=(grid,),
        in_specs=[chunk_spec] * 14 + [
            pl.BlockSpec((_RB, _NAP), lambda i: (i, 0)),
        ],
        out_specs=(
            pl.BlockSpec((_RB, _NA), lambda i: (i, 0)),
            pl.BlockSpec((_RB, _K), lambda i: (i, 0)),
        ),
        out_shape=(
            jax.ShapeDtypeStruct((_N, _NA), jnp.float32),
            jax.ShapeDtypeStruct((_N, _K), jnp.float32),
        ),
    )(*pchunks, *aggp, q)


def kernel(x, edges, A_init, W, b, temperature):
    del edges
    anchor_idx = jax.random.choice(jax.random.key(1), _N, shape=(_NA,),
                                   replace=False)
    q = jax.random.uniform(jax.random.key(2), (_N, _NA), dtype=jnp.float32)
    q = q + 1e-12
    qp = jnp.pad(q, ((0, 0), (0, _NAP - _NA)), constant_values=0.5)

    ax = jnp.zeros((_NAP, _DF), jnp.float32).at[:_NA].set(x[anchor_idx])
    t = jnp.reshape(temperature, (1,)).astype(jnp.float32)
    b2 = jnp.reshape(b, (1, _DF))

    xemb, p0, p1, p2, p3 = _emb_prob(x, ax, W, b2, t)

    npad_e = _EPAD - _E
    srcp = jnp.concatenate(
        [A_init[0], jnp.full((npad_e,), _N, jnp.int32)])
    dstp = jnp.concatenate(
        [A_init[1], jnp.zeros((npad_e,), jnp.int32)])
    zrows = jnp.zeros((_NPAD, _CW), jnp.float32)
    ones128 = jnp.ones((_BE, _CW), jnp.float32)
    aggp = _sc_scatter((p0, p1, p2, p3), srcp, dstp, zrows, ones128)

    ehat, lp = _topk((p0, p1, p2, p3), aggp, qp)
    return (xemb, ehat, lp)
